# baseline (speedup 1.0000x reference)
import jax
import jax.numpy as jnp
from jax.experimental import pallas as pl

B, Cg, H, W = 4, 5, 384, 384
Cr = 384
P = 16
Hf = H // P
Wf = W // P
M = B * Hf * Wf
N_MEM = 9216
TILE_M = 576
MB = Hf * Wf

_NT = (((1,), (1,)), ((), ()))


def _prep_kernel(img_ref, g_ref, r_ref, bank_ref, p_ref, geo8_ref, bn_ref):
    f32 = jnp.float32
    for i in range(Hf):
        x = jnp.concatenate(
            [img_ref[0, c, P * i:P * (i + 1), :] for c in range(3)],
            axis=0).astype(jnp.bfloat16)
        t = x.T.reshape(Wf, P, 48)
        p_ref[Wf * i:Wf * (i + 1), :] = jnp.concatenate(
            [t[:, pw, :] for pw in range(P)], axis=1)
    r = r_ref[...]
    r_tile = jnp.broadcast_to(r[None, :, :], (Hf, Wf, W)).reshape(MB, W)
    cols = []
    for c in range(Cg):
        t1 = jnp.dot(r, g_ref[0, c], preferred_element_type=f32)
        e = jnp.broadcast_to(t1[:, None, :], (Hf, Wf, W)).reshape(MB, W)
        cols.append(jnp.sum(e * r_tile, axis=1, keepdims=True))
    cols.append(jnp.zeros((MB, 8 - Cg), dtype=f32))
    geo8_ref[...] = jnp.concatenate(cols, axis=1)
    @pl.when(pl.program_id(0) == 0)
    def _():
        bk = bank_ref[...].astype(f32)
        ones = jnp.ones((1, Cr), dtype=f32)
        bn_ref[...] = jax.lax.dot_general(
            ones, bk * bk, _NT, preferred_element_type=f32)


def _main_kernel(p_ref, w1_ref, b1_ref, w2a_ref, w2g_ref, b2_ref,
                 geo_ref, bank_ref, bn_ref, sp_ref, sg_ref):
    bf16 = jnp.bfloat16
    feat = jnp.dot(p_ref[...], w1_ref[...], preferred_element_type=jnp.float32)
    feat = jnp.maximum(feat + b1_ref[...], 0.0)
    geo = geo_ref[...]
    flat = (jnp.dot(feat.astype(bf16), w2a_ref[...],
                    preferred_element_type=jnp.float32)
            + jnp.dot(geo.astype(bf16), w2g_ref[...],
                      preferred_element_type=jnp.float32)
            + b2_ref[...])
    fn = jnp.sum(flat * flat, axis=1, keepdims=True)
    flat_m2 = (-2.0 * flat).astype(bf16)
    prod = jax.lax.dot_general(
        flat_m2, bank_ref[...], _NT,
        preferred_element_type=jnp.float32)
    t = prod + bn_ref[...]
    dmin = jnp.min(t, axis=1, keepdims=True) + fn
    sp = jnp.sqrt(jnp.maximum(dmin, 0.0) + 1e-12)
    base = (0.5 * geo[:, 3:4] + 0.25 * (1.0 - geo[:, 2:3])
            + 0.25 * geo[:, 4:5])
    wgt = 1.0 + jax.nn.sigmoid(4.0 * (base - 0.5))
    sp_ref[...] = sp
    sg_ref[...] = sp * wgt


def kernel(image, geometry_map, bb_w, bb_b, fu_w, fu_b, memory_bank):
    f32 = jnp.float32
    bf16 = jnp.bfloat16
    w1 = bb_w.transpose(3, 1, 2, 0).reshape(3 * P * P, Cr).astype(bf16)
    b1 = bb_b.reshape(1, Cr)
    w2 = fu_w[:, :, 0, 0]
    w2a = w2[:, :Cr].T.astype(bf16)
    w2g = jnp.pad(w2[:, Cr:].T, ((0, 8 - Cg), (0, 0))).astype(bf16)
    b2 = fu_b.reshape(1, Cr)
    bank_bf = memory_bank.astype(bf16)
    r_op = jax.image.resize(jnp.eye(H, dtype=f32), (Hf, H), method='bilinear')

    patches, geo8, bn = pl.pallas_call(
        _prep_kernel,
        grid=(B,),
        in_specs=[
            pl.BlockSpec((1, 3, H, W), lambda i: (i, 0, 0, 0)),
            pl.BlockSpec((1, Cg, H, W), lambda i: (i, 0, 0, 0)),
            pl.BlockSpec((Hf, H), lambda i: (0, 0)),
            pl.BlockSpec((N_MEM, Cr), lambda i: (0, 0)),
        ],
        out_specs=[
            pl.BlockSpec((MB, 3 * P * P), lambda i: (i, 0)),
            pl.BlockSpec((MB, 8), lambda i: (i, 0)),
            pl.BlockSpec((1, N_MEM), lambda i: (0, 0)),
        ],
        out_shape=[
            jax.ShapeDtypeStruct((M, 3 * P * P), bf16),
            jax.ShapeDtypeStruct((M, 8), f32),
            jax.ShapeDtypeStruct((1, N_MEM), f32),
        ],
    )(image, geometry_map, r_op, bank_bf)

    grid = (M // TILE_M,)
    sp, sg = pl.pallas_call(
        _main_kernel,
        grid=grid,
        in_specs=[
            pl.BlockSpec((TILE_M, 3 * P * P), lambda i: (i, 0)),
            pl.BlockSpec((3 * P * P, Cr), lambda i: (0, 0)),
            pl.BlockSpec((1, Cr), lambda i: (0, 0)),
            pl.BlockSpec((Cr, Cr), lambda i: (0, 0)),
            pl.BlockSpec((8, Cr), lambda i: (0, 0)),
            pl.BlockSpec((1, Cr), lambda i: (0, 0)),
            pl.BlockSpec((TILE_M, 8), lambda i: (i, 0)),
            pl.BlockSpec((N_MEM, Cr), lambda i: (0, 0)),
            pl.BlockSpec((1, N_MEM), lambda i: (0, 0)),
        ],
        out_specs=[
            pl.BlockSpec((TILE_M, 1), lambda i: (i, 0)),
            pl.BlockSpec((TILE_M, 1), lambda i: (i, 0)),
        ],
        out_shape=[
            jax.ShapeDtypeStruct((M, 1), f32),
            jax.ShapeDtypeStruct((M, 1), f32),
        ],
    )(patches, w1, b1, w2a, w2g, b2, geo8, bank_bf, bn)

    score_plain = sp.reshape(B, Hf, Wf)
    score_geo = sg.reshape(B, Hf, Wf)
    return (score_plain, score_geo)


# speedup vs baseline: 1.0334x; 1.0334x over previous
import jax
import jax.numpy as jnp
from jax.experimental import pallas as pl

B, Cg, H, W = 4, 5, 384, 384
Cr = 384
P = 16
Hf = H // P
Wf = W // P
M = B * Hf * Wf
N_MEM = 9216
TILE_M = 576
MB = Hf * Wf

_NT = (((1,), (1,)), ((), ()))

R_OP = jax.image.resize(jnp.eye(H, dtype=jnp.float32), (Hf, H),
                        method='bilinear')


def _prep_kernel(img_ref, g_ref, r_ref, bank_ref, p_ref, geo8_ref, bn_ref):
    f32 = jnp.float32
    for i in range(Hf):
        x = jnp.concatenate(
            [img_ref[0, c, P * i:P * (i + 1), :] for c in range(3)],
            axis=0)
        t = x.T.reshape(Wf, P, 48)
        p_ref[Wf * i:Wf * (i + 1), :] = jnp.concatenate(
            [t[:, pw, :] for pw in range(P)], axis=1)
    r = r_ref[...]
    r_tile = jnp.broadcast_to(r[None, :, :], (Hf, Wf, W)).reshape(MB, W)
    cols = []
    for c in range(Cg):
        t1 = jnp.dot(r, g_ref[0, c], preferred_element_type=f32)
        e = jnp.broadcast_to(t1[:, None, :], (Hf, Wf, W)).reshape(MB, W)
        cols.append(jnp.sum(e * r_tile, axis=1, keepdims=True))
    cols.append(jnp.zeros((MB, 8 - Cg), dtype=f32))
    geo8_ref[...] = jnp.concatenate(cols, axis=1)
    @pl.when(pl.program_id(0) == 0)
    def _():
        bk = bank_ref[...].astype(f32)
        ones = jnp.ones((1, Cr), dtype=f32)
        bn_ref[...] = jax.lax.dot_general(
            ones, bk * bk, _NT, preferred_element_type=f32)


def _main_kernel(p_ref, w1_ref, b1_ref, w2a_ref, w2g_ref, b2_ref,
                 geo_ref, bank_ref, bn_ref, sp_ref, sg_ref):
    bf16 = jnp.bfloat16
    feat = jnp.dot(p_ref[...], w1_ref[...], preferred_element_type=jnp.float32)
    feat = jnp.maximum(feat + b1_ref[...], 0.0)
    geo = geo_ref[...]
    flat = (jnp.dot(feat.astype(bf16), w2a_ref[...],
                    preferred_element_type=jnp.float32)
            + jnp.dot(geo.astype(bf16), w2g_ref[...],
                      preferred_element_type=jnp.float32)
            + b2_ref[...])
    fn = jnp.sum(flat * flat, axis=1, keepdims=True)
    flat_m2 = (-2.0 * flat).astype(bf16)
    prod = jax.lax.dot_general(
        flat_m2, bank_ref[...], _NT,
        preferred_element_type=jnp.float32)
    t = prod + bn_ref[...]
    dmin = jnp.min(t, axis=1, keepdims=True) + fn
    sp = jnp.sqrt(jnp.maximum(dmin, 0.0) + 1e-12)
    base = (0.5 * geo[:, 3:4] + 0.25 * (1.0 - geo[:, 2:3])
            + 0.25 * geo[:, 4:5])
    wgt = 1.0 + jax.nn.sigmoid(4.0 * (base - 0.5))
    sp_ref[...] = sp
    sg_ref[...] = sp * wgt


def kernel(image, geometry_map, bb_w, bb_b, fu_w, fu_b, memory_bank):
    f32 = jnp.float32
    bf16 = jnp.bfloat16
    img_bf = image.astype(bf16)
    w1 = bb_w.transpose(3, 1, 2, 0).reshape(3 * P * P, Cr).astype(bf16)
    b1 = bb_b.reshape(1, Cr)
    w2 = fu_w[:, :, 0, 0]
    w2a = w2[:, :Cr].T.astype(bf16)
    w2g = jnp.pad(w2[:, Cr:].T, ((0, 8 - Cg), (0, 0))).astype(bf16)
    b2 = fu_b.reshape(1, Cr)
    bank_bf = memory_bank.astype(bf16)
    r_op = R_OP

    patches, geo8, bn = pl.pallas_call(
        _prep_kernel,
        grid=(B,),
        in_specs=[
            pl.BlockSpec((1, 3, H, W), lambda i: (i, 0, 0, 0)),
            pl.BlockSpec((1, Cg, H, W), lambda i: (i, 0, 0, 0)),
            pl.BlockSpec((Hf, H), lambda i: (0, 0)),
            pl.BlockSpec((N_MEM, Cr), lambda i: (0, 0)),
        ],
        out_specs=[
            pl.BlockSpec((MB, 3 * P * P), lambda i: (i, 0)),
            pl.BlockSpec((MB, 8), lambda i: (i, 0)),
            pl.BlockSpec((1, N_MEM), lambda i: (0, 0)),
        ],
        out_shape=[
            jax.ShapeDtypeStruct((M, 3 * P * P), bf16),
            jax.ShapeDtypeStruct((M, 8), f32),
            jax.ShapeDtypeStruct((1, N_MEM), f32),
        ],
    )(img_bf, geometry_map, r_op, bank_bf)

    grid = (M // TILE_M,)
    sp, sg = pl.pallas_call(
        _main_kernel,
        grid=grid,
        in_specs=[
            pl.BlockSpec((TILE_M, 3 * P * P), lambda i: (i, 0)),
            pl.BlockSpec((3 * P * P, Cr), lambda i: (0, 0)),
            pl.BlockSpec((1, Cr), lambda i: (0, 0)),
            pl.BlockSpec((Cr, Cr), lambda i: (0, 0)),
            pl.BlockSpec((8, Cr), lambda i: (0, 0)),
            pl.BlockSpec((1, Cr), lambda i: (0, 0)),
            pl.BlockSpec((TILE_M, 8), lambda i: (i, 0)),
            pl.BlockSpec((N_MEM, Cr), lambda i: (0, 0)),
            pl.BlockSpec((1, N_MEM), lambda i: (0, 0)),
        ],
        out_specs=[
            pl.BlockSpec((TILE_M, 1), lambda i: (i, 0)),
            pl.BlockSpec((TILE_M, 1), lambda i: (i, 0)),
        ],
        out_shape=[
            jax.ShapeDtypeStruct((M, 1), f32),
            jax.ShapeDtypeStruct((M, 1), f32),
        ],
    )(patches, w1, b1, w2a, w2g, b2, geo8, bank_bf, bn)

    score_plain = sp.reshape(B, Hf, Wf)
    score_geo = sg.reshape(B, Hf, Wf)
    return (score_plain, score_geo)


# speedup vs baseline: 1.0359x; 1.0025x over previous
import jax
import jax.numpy as jnp
from jax.experimental import pallas as pl

B, Cg, H, W = 4, 5, 384, 384
Cr = 384
P = 16
Hf = H // P
Wf = W // P
M = B * Hf * Wf
N_MEM = 9216
TILE_M = 576
MB = Hf * Wf
NCH = 2
NC = N_MEM // NCH

_NT = (((1,), (1,)), ((), ()))

R_OP = jax.image.resize(jnp.eye(H, dtype=jnp.float32), (Hf, H),
                        method='bilinear')


def _prep_kernel(img_ref, g_ref, r_ref, bank_ref, p_ref, geo8_ref, bn_ref):
    f32 = jnp.float32
    for i in range(Hf):
        x = jnp.concatenate(
            [img_ref[0, c, P * i:P * (i + 1), :] for c in range(3)],
            axis=0)
        t = x.T.reshape(Wf, P, 48)
        p_ref[Wf * i:Wf * (i + 1), :] = jnp.concatenate(
            [t[:, pw, :] for pw in range(P)], axis=1)
    r = r_ref[...]
    r_tile = jnp.broadcast_to(r[None, :, :], (Hf, Wf, W)).reshape(MB, W)
    cols = []
    for c in range(Cg):
        t1 = jnp.dot(r, g_ref[0, c], preferred_element_type=f32)
        e = jnp.broadcast_to(t1[:, None, :], (Hf, Wf, W)).reshape(MB, W)
        cols.append(jnp.sum(e * r_tile, axis=1, keepdims=True))
    cols.append(jnp.zeros((MB, 8 - Cg), dtype=f32))
    geo8_ref[...] = jnp.concatenate(cols, axis=1)
    @pl.when(pl.program_id(0) == 0)
    def _():
        bk = bank_ref[...].astype(f32)
        ones = jnp.ones((1, Cr), dtype=f32)
        bn_ref[...] = jax.lax.dot_general(
            ones, bk * bk, _NT, preferred_element_type=f32)


def _main_kernel(p_ref, w1_ref, b1_ref, w2a_ref, w2g_ref, b2_ref,
                 geo_ref, bank_ref, bn_ref, sp_ref, sg_ref):
    bf16 = jnp.bfloat16
    feat = jnp.dot(p_ref[...], w1_ref[...], preferred_element_type=jnp.float32)
    feat = jnp.maximum(feat + b1_ref[...], 0.0)
    geo = geo_ref[...]
    flat = (jnp.dot(feat.astype(bf16), w2a_ref[...],
                    preferred_element_type=jnp.float32)
            + jnp.dot(geo.astype(bf16), w2g_ref[...],
                      preferred_element_type=jnp.float32)
            + b2_ref[...])
    fn = jnp.sum(flat * flat, axis=1, keepdims=True)
    flat_m2 = (-2.0 * flat).astype(bf16)
    dmin = None
    for n in range(NCH):
        prod = jax.lax.dot_general(
            flat_m2, bank_ref[NC * n:NC * (n + 1), :], _NT,
            preferred_element_type=jnp.float32)
        t = prod + bn_ref[:, NC * n:NC * (n + 1)]
        m = jnp.min(t, axis=1, keepdims=True)
        dmin = m if dmin is None else jnp.minimum(dmin, m)
    dmin = dmin + fn
    sp = jnp.sqrt(jnp.maximum(dmin, 0.0) + 1e-12)
    base = (0.5 * geo[:, 3:4] + 0.25 * (1.0 - geo[:, 2:3])
            + 0.25 * geo[:, 4:5])
    wgt = 1.0 + jax.nn.sigmoid(4.0 * (base - 0.5))
    sp_ref[...] = sp
    sg_ref[...] = sp * wgt


def kernel(image, geometry_map, bb_w, bb_b, fu_w, fu_b, memory_bank):
    f32 = jnp.float32
    bf16 = jnp.bfloat16
    img_bf = image.astype(bf16)
    w1 = bb_w.transpose(3, 1, 2, 0).reshape(3 * P * P, Cr).astype(bf16)
    b1 = bb_b.reshape(1, Cr)
    w2 = fu_w[:, :, 0, 0]
    w2a = w2[:, :Cr].T.astype(bf16)
    w2g = jnp.pad(w2[:, Cr:].T, ((0, 8 - Cg), (0, 0))).astype(bf16)
    b2 = fu_b.reshape(1, Cr)
    bank_bf = memory_bank.astype(bf16)
    r_op = R_OP

    patches, geo8, bn = pl.pallas_call(
        _prep_kernel,
        grid=(B,),
        in_specs=[
            pl.BlockSpec((1, 3, H, W), lambda i: (i, 0, 0, 0)),
            pl.BlockSpec((1, Cg, H, W), lambda i: (i, 0, 0, 0)),
            pl.BlockSpec((Hf, H), lambda i: (0, 0)),
            pl.BlockSpec((N_MEM, Cr), lambda i: (0, 0)),
        ],
        out_specs=[
            pl.BlockSpec((MB, 3 * P * P), lambda i: (i, 0)),
            pl.BlockSpec((MB, 8), lambda i: (i, 0)),
            pl.BlockSpec((1, N_MEM), lambda i: (0, 0)),
        ],
        out_shape=[
            jax.ShapeDtypeStruct((M, 3 * P * P), bf16),
            jax.ShapeDtypeStruct((M, 8), f32),
            jax.ShapeDtypeStruct((1, N_MEM), f32),
        ],
    )(img_bf, geometry_map, r_op, bank_bf)

    grid = (M // TILE_M,)
    sp, sg = pl.pallas_call(
        _main_kernel,
        grid=grid,
        in_specs=[
            pl.BlockSpec((TILE_M, 3 * P * P), lambda i: (i, 0)),
            pl.BlockSpec((3 * P * P, Cr), lambda i: (0, 0)),
            pl.BlockSpec((1, Cr), lambda i: (0, 0)),
            pl.BlockSpec((Cr, Cr), lambda i: (0, 0)),
            pl.BlockSpec((8, Cr), lambda i: (0, 0)),
            pl.BlockSpec((1, Cr), lambda i: (0, 0)),
            pl.BlockSpec((TILE_M, 8), lambda i: (i, 0)),
            pl.BlockSpec((N_MEM, Cr), lambda i: (0, 0)),
            pl.BlockSpec((1, N_MEM), lambda i: (0, 0)),
        ],
        out_specs=[
            pl.BlockSpec((TILE_M, 1), lambda i: (i, 0)),
            pl.BlockSpec((TILE_M, 1), lambda i: (i, 0)),
        ],
        out_shape=[
            jax.ShapeDtypeStruct((M, 1), f32),
            jax.ShapeDtypeStruct((M, 1), f32),
        ],
    )(patches, w1, b1, w2a, w2g, b2, geo8, bank_bf, bn)

    score_plain = sp.reshape(B, Hf, Wf)
    score_geo = sg.reshape(B, Hf, Wf)
    return (score_plain, score_geo)
